# baseline (speedup 1.0000x reference)
import functools

import jax
import jax.numpy as jnp
from jax import lax
from jax.experimental import pallas as pl
from jax.experimental.pallas import tpu as pltpu
from jax.experimental.pallas import tpu_sc as plsc

_NU = 10000
_NV = 10000
_NE = 320000
_D = 128

_NC = 2
_NS = 16
_NW = _NC * _NS
_EPW = _NE // _NW
_CHUNK = 80
_NCHUNK = _EPW // _CHUNK
_NSEG = 10240
_SEG_PT = _NSEG // _NS
_ZROWS = 128

_sc_mesh = plsc.VectorSubcoreMesh(
    core_axis_name="c", subcore_axis_name="s",
    num_cores=_NC, num_subcores=_NS)


def _ln(x, g, b):
    m = jnp.mean(x, axis=-1, keepdims=True)
    d = x - m
    v = jnp.mean(d * d, axis=-1, keepdims=True)
    return d * lax.rsqrt(v + 1e-5) * g + b


def _prep_body(v_ref, c_ref, vg, vb, cg, cb, wl, bl, wr, vt_ref, ct_ref):
    x = _ln(v_ref[...], vg[...], vb[...])
    vt_ref[...] = (
        jnp.dot(x, wl[...], preferred_element_type=jnp.float32) + bl[...])
    y = _ln(c_ref[...], cg[...], cb[...])
    ct_ref[...] = jnp.dot(y, wr[...], preferred_element_type=jnp.float32)


def _prep(v0, c0, vg, vb, cg, cb, wl, bl, wr):
    return pl.pallas_call(
        _prep_body,
        out_shape=(jax.ShapeDtypeStruct((_NU, _D), jnp.float32),
                   jax.ShapeDtypeStruct((_NV, _D), jnp.float32)),
    )(v0, c0, vg.reshape(1, _D), vb.reshape(1, _D), cg.reshape(1, _D),
      cb.reshape(1, _D), wl, bl.reshape(1, _D), wr)


_RB = 10000


def _joint_body(e_ref, g_ref, leg, leb, ljg, ljb, jtg, jtb, we, wj, bj,
                o_ref):
    e = jnp.dot(_ln(e_ref[...], leg[...], leb[...]), we[...],
                preferred_element_type=jnp.float32)
    j = jnp.maximum(e + g_ref[...], 0.0)
    j = jnp.dot(_ln(j, ljg[...], ljb[...]), wj[...],
                preferred_element_type=jnp.float32) + bj[...]
    o_ref[...] = _ln(j, jtg[...], jtb[...])


def _joint(edge_emb, gsum, leg, leb, ljg, ljb, jtg, jtb, we, wj, bj):
    row = pl.BlockSpec((_RB, _D), lambda i: (i, 0))
    par = pl.BlockSpec((1, _D), lambda i: (0, 0))
    mat = pl.BlockSpec((_D, _D), lambda i: (0, 0))
    return pl.pallas_call(
        _joint_body,
        grid=(_NE // _RB,),
        in_specs=[row, row, par, par, par, par, par, par, mat, mat, par],
        out_specs=row,
        out_shape=jax.ShapeDtypeStruct((_NE, _D), jnp.float32),
    )(edge_emb, gsum, leg.reshape(1, _D), leb.reshape(1, _D),
      ljg.reshape(1, _D), ljb.reshape(1, _D), jtg.reshape(1, _D),
      jtb.reshape(1, _D), we, wj, bj.reshape(1, _D))


def _merge_body(t_ref, p_ref, wa, wb, bm, gg, gb, o_ref):
    n = t_ref.shape[0]
    agg = p_ref[0, 0:n] + p_ref[1, 0:n]
    h = (jnp.dot(t_ref[...], wa[...], preferred_element_type=jnp.float32)
         + jnp.dot(agg, wb[...], preferred_element_type=jnp.float32)
         + bm[...])
    o_ref[...] = t_ref[...] + _ln(jnp.maximum(h, 0.0), gg[...], gb[...])


def _merge(tbl, parts, wa, wb, bm, gg, gb):
    return pl.pallas_call(
        _merge_body,
        out_shape=jax.ShapeDtypeStruct((tbl.shape[0], _D), jnp.float32),
    )(tbl, parts, wa, wb, bm.reshape(1, _D), gg.reshape(1, _D),
      gb.reshape(1, _D))


def _merge2_body(t_ref, p_ref, wa, wb, bm, gg, gb, cg, cb, wr,
                 o_ref, o2_ref):
    n = t_ref.shape[0]
    agg = p_ref[0, 0:n] + p_ref[1, 0:n]
    h = (jnp.dot(t_ref[...], wa[...], preferred_element_type=jnp.float32)
         + jnp.dot(agg, wb[...], preferred_element_type=jnp.float32)
         + bm[...])
    out = t_ref[...] + _ln(jnp.maximum(h, 0.0), gg[...], gb[...])
    o_ref[...] = out
    o2_ref[...] = jnp.dot(_ln(out, cg[...], cb[...]), wr[...],
                          preferred_element_type=jnp.float32)


def _merge2(tbl, parts, wa, wb, bm, gg, gb, cg, cb, wr):
    n = tbl.shape[0]
    return pl.pallas_call(
        _merge2_body,
        out_shape=(jax.ShapeDtypeStruct((n, _D), jnp.float32),
                   jax.ShapeDtypeStruct((n, _D), jnp.float32)),
    )(tbl, parts, wa, wb, bm.reshape(1, _D), gg.reshape(1, _D),
      gb.reshape(1, _D), cg.reshape(1, _D), cb.reshape(1, _D), wr)


@functools.partial(
    pl.kernel,
    out_type=jax.ShapeDtypeStruct((_NE, _D), jnp.float32),
    mesh=_sc_mesh,
    scratch_types=[
        pltpu.VMEM((_CHUNK,), jnp.int32),
        pltpu.VMEM((_CHUNK,), jnp.int32),
        pltpu.VMEM((_CHUNK,), jnp.int32),
        pltpu.VMEM((_CHUNK,), jnp.int32),
        pltpu.VMEM((_CHUNK, _D), jnp.float32),
        pltpu.VMEM((_CHUNK, _D), jnp.float32),
        pltpu.VMEM((_CHUNK, _D), jnp.float32),
        pltpu.VMEM((_CHUNK, _D), jnp.float32),
        pltpu.SemaphoreType.DMA,
        pltpu.SemaphoreType.DMA,
        pltpu.SemaphoreType.DMA,
        pltpu.SemaphoreType.DMA,
        pltpu.SemaphoreType.DMA,
        pltpu.SemaphoreType.DMA,
    ],
)
def _gather_add(vt, ct, eu, ev, out,
                idxu0, idxu1, idxv0, idxv1, a0, a1, b0, b1,
                si0, si1, sg0, sg1, so0, so1):
    sid = lax.axis_index("s")
    wid = sid * _NC + lax.axis_index("c")
    IU, IV, A, B = [idxu0, idxu1], [idxv0, idxv1], [a0, a1], [b0, b1]
    SI, SG, SO = [si0, si1], [sg0, sg1], [so0, so1]

    def base_of(c):
        return pl.multiple_of(wid * _EPW + c * _CHUNK, _CHUNK)

    def fire_idx(c, p):
        b = base_of(c)
        pltpu.async_copy(eu.at[pl.ds(b, _CHUNK)], IU[p], SI[p])
        pltpu.async_copy(ev.at[pl.ds(b, _CHUNK)], IV[p], SI[p])

    def wait_idx(p):
        pltpu.make_async_copy(eu.at[pl.ds(0, _CHUNK)], IU[p], SI[p]).wait()
        pltpu.make_async_copy(ev.at[pl.ds(0, _CHUNK)], IV[p], SI[p]).wait()

    def fire_gather(p):
        pltpu.async_copy(vt.at[IU[p]], A[p], SG[p])
        pltpu.async_copy(ct.at[IV[p]], B[p], SG[p])

    def wait_gather(p):
        pltpu.make_async_copy(vt.at[IU[p]], A[p], SG[p]).wait()
        pltpu.make_async_copy(ct.at[IV[p]], B[p], SG[p]).wait()

    def fire_out(c, p):
        pltpu.async_copy(A[p], out.at[pl.ds(base_of(c), _CHUNK)], SO[p])

    def wait_out(p):
        pltpu.make_async_copy(A[p], out.at[pl.ds(0, _CHUNK)], SO[p]).wait()

    def add(p):
        def addrow(k, carry):
            for u in range(2):
                r = 2 * k + u
                for q in range(_D // 16):
                    s = pl.ds(q * 16, 16)
                    A[p][r, s] = A[p][r, s] + B[p][r, s]
            return carry
        lax.fori_loop(0, _CHUNK // 2, addrow, 0)

    fire_idx(0, 0)
    fire_idx(1, 1)
    wait_idx(0)
    fire_gather(0)
    wait_gather(0); fire_idx(2, 0); wait_idx(1); fire_gather(1)
    add(0); fire_out(0, 0)
    wait_gather(1); fire_idx(3, 1); wait_idx(0); wait_out(0)
    fire_gather(0); add(1); fire_out(1, 1)

    def pair(k, carry):
        c0 = 2 * k
        wait_gather(0); fire_idx(c0 + 2, 0); wait_idx(1); wait_out(1)
        fire_gather(1); add(0); fire_out(c0, 0)
        wait_gather(1); fire_idx(c0 + 3, 1); wait_idx(0); wait_out(0)
        fire_gather(0); add(1); fire_out(c0 + 1, 1)
        return carry

    lax.fori_loop(1, 61, pair, 0)

    wait_gather(0); fire_idx(124, 0); wait_idx(1); wait_out(1)
    fire_gather(1); add(0); fire_out(122, 0)
    wait_gather(1); wait_idx(0); wait_out(0)
    fire_gather(0); add(1); fire_out(123, 1)
    wait_gather(0); add(0); fire_out(124, 0)
    wait_out(1); wait_out(0)


@functools.partial(
    pl.kernel,
    out_type=jax.ShapeDtypeStruct((_NC, _NSEG, _D), jnp.float32),
    mesh=_sc_mesh,
    scratch_types=[
        pltpu.VMEM((_CHUNK,), jnp.int32),
        pltpu.VMEM((_CHUNK,), jnp.int32),
        pltpu.VMEM((_CHUNK, _D), jnp.float32),
        pltpu.VMEM((_CHUNK, _D), jnp.float32),
        pltpu.VMEM((_ZROWS, _D), jnp.float32),
        pltpu.VMEM_SHARED((_NSEG, _D), jnp.float32),
        pltpu.SemaphoreType.DMA,
        pltpu.SemaphoreType.DMA,
        pltpu.SemaphoreType.DMA,
        pltpu.SemaphoreType.DMA,
    ],
)
def _seg_sum(joint, idx, out, i0, i1, r0_, r1_, zbuf, acc, si0, si1, ss0, ss1):
    cid = lax.axis_index("c")
    sid = lax.axis_index("s")
    wid = sid * _NC + cid
    IX, R = [i0, i1], [r0_, r1_]
    SI, SS = [si0, si1], [ss0, ss1]

    def zrow(r, carry):
        for q in range(_D // 16):
            zbuf[r, pl.ds(q * 16, 16)] = jnp.zeros((16,), jnp.float32)
        return carry

    lax.fori_loop(0, _ZROWS, zrow, 0)
    for k in range(_SEG_PT // _ZROWS):
        pltpu.sync_copy(zbuf, acc.at[pl.ds(sid * _SEG_PT + k * _ZROWS, _ZROWS)])
    plsc.subcore_barrier()

    def base_of(c):
        return pl.multiple_of(wid * _EPW + c * _CHUNK, _CHUNK)

    def fire_in(c, p):
        b = base_of(c)
        pltpu.async_copy(idx.at[pl.ds(b, _CHUNK)], IX[p], SI[p])
        pltpu.async_copy(joint.at[pl.ds(b, _CHUNK)], R[p], SI[p])

    def wait_in(p):
        pltpu.make_async_copy(idx.at[pl.ds(0, _CHUNK)], IX[p], SI[p]).wait()
        pltpu.make_async_copy(joint.at[pl.ds(0, _CHUNK)], R[p], SI[p]).wait()

    def fire_scat(p):
        pltpu.async_copy(R[p], acc.at[IX[p]], SS[p], add=True)

    def wait_scat(p):
        pltpu.make_async_copy(R[p], acc.at[IX[p]], SS[p]).wait()

    fire_in(0, 0)
    wait_in(0); fire_scat(0); fire_in(1, 1)

    def pair(k, carry):
        c = 2 * k + 1
        wait_in(1); fire_scat(1); wait_scat(0); fire_in(c + 1, 0)
        wait_in(0); fire_scat(0); wait_scat(1); fire_in(c + 2, 1)
        return carry

    lax.fori_loop(0, 61, pair, 0)

    wait_in(1); fire_scat(1); wait_scat(0); fire_in(124, 0)
    wait_in(0); fire_scat(0); wait_scat(1); wait_scat(0)

    plsc.subcore_barrier()
    for k in range(_SEG_PT // _ZROWS):
        r0 = sid * _SEG_PT + k * _ZROWS
        pltpu.sync_copy(acc.at[pl.ds(r0, _ZROWS)], zbuf)
        pltpu.sync_copy(zbuf, out.at[cid, pl.ds(r0, _ZROWS)])


def kernel(variable_emb, edge_emb, constraint_emb, e_u, e_v, W_left, b_left,
           W_edge, W_right, W_join, b_join, W_merge, b_merge, ln_var_g,
           ln_var_b, ln_edge_g, ln_edge_b, ln_con_g, ln_con_b, ln_joint_g,
           ln_joint_b, ln_join_g, ln_join_b, ln_merge_g, ln_merge_b):
    wm_a = W_merge[:_D]
    wm_b = W_merge[_D:]

    vt, ct1 = _prep(variable_emb, constraint_emb, ln_var_g, ln_var_b,
                    ln_con_g, ln_con_b, W_left, b_left, W_right)

    g1 = _gather_add(vt, ct1, e_u, e_v)
    j1 = _joint(edge_emb, g1, ln_edge_g, ln_edge_b, ln_join_g, ln_join_b,
                ln_joint_g, ln_joint_b, W_edge, W_join, b_join)
    p1 = _seg_sum(j1, e_v)
    c_new, ct2 = _merge2(ct1, p1, wm_a, wm_b, b_merge, ln_merge_g,
                         ln_merge_b, ln_con_g, ln_con_b, W_right)

    g2 = _gather_add(vt, ct2, e_u, e_v)
    j2 = _joint(edge_emb, g2, ln_edge_g, ln_edge_b, ln_join_g, ln_join_b,
                ln_joint_g, ln_joint_b, W_edge, W_join, b_join)
    p2 = _seg_sum(j2, e_u)
    v_new = _merge(vt, p2, wm_a, wm_b, b_merge, ln_merge_g, ln_merge_b)

    return (v_new, c_new)


# speedup vs baseline: 1.0773x; 1.0773x over previous
import functools

import jax
import jax.numpy as jnp
from jax import lax
from jax.experimental import pallas as pl
from jax.experimental.pallas import tpu as pltpu
from jax.experimental.pallas import tpu_sc as plsc

_NU = 10000
_NV = 10000
_NE = 320000
_D = 128

_NC = 2
_NS = 16
_NW = _NC * _NS
_EPW = _NE // _NW
_CHUNK = 80
_NCHUNK = _EPW // _CHUNK
_NSEG = 10240
_SEG_PT = _NSEG // _NS
_ZROWS = 128

_sc_mesh = plsc.VectorSubcoreMesh(
    core_axis_name="c", subcore_axis_name="s",
    num_cores=_NC, num_subcores=_NS)


def _ln(x, g, b):
    m = jnp.mean(x, axis=-1, keepdims=True)
    d = x - m
    v = jnp.mean(d * d, axis=-1, keepdims=True)
    return d * lax.rsqrt(v + 1e-5) * g + b


def _prep_body(v_ref, c_ref, vg, vb, cg, cb, wl, bl, wr, vt_ref, ct_ref):
    x = _ln(v_ref[...], vg[...], vb[...])
    vt_ref[...] = (
        jnp.dot(x, wl[...], preferred_element_type=jnp.float32) + bl[...])
    y = _ln(c_ref[...], cg[...], cb[...])
    ct_ref[...] = jnp.dot(y, wr[...], preferred_element_type=jnp.float32)


def _prep(v0, c0, vg, vb, cg, cb, wl, bl, wr):
    return pl.pallas_call(
        _prep_body,
        out_shape=(jax.ShapeDtypeStruct((_NU, _D), jnp.float32),
                   jax.ShapeDtypeStruct((_NV, _D), jnp.float32)),
    )(v0, c0, vg.reshape(1, _D), vb.reshape(1, _D), cg.reshape(1, _D),
      cb.reshape(1, _D), wl, bl.reshape(1, _D), wr)


_RB = 10000


def _edge_body(e_ref, leg, leb, we, o_ref):
    o_ref[...] = jnp.dot(_ln(e_ref[...], leg[...], leb[...]), we[...],
                         preferred_element_type=jnp.float32)


def _edge_mm(edge_emb, leg, leb, we):
    row = pl.BlockSpec((_RB, _D), lambda i: (i, 0))
    par = pl.BlockSpec((1, _D), lambda i: (0, 0))
    mat = pl.BlockSpec((_D, _D), lambda i: (0, 0))
    return pl.pallas_call(
        _edge_body,
        grid=(_NE // _RB,),
        in_specs=[row, par, par, mat],
        out_specs=row,
        out_shape=jax.ShapeDtypeStruct((_NE, _D), jnp.float32),
    )(edge_emb, leg.reshape(1, _D), leb.reshape(1, _D), we)


def _joint_body(et_ref, g_ref, ljg, ljb, jtg, jtb, wj, bj, o_ref):
    j = jnp.maximum(et_ref[...] + g_ref[...], 0.0)
    j = jnp.dot(_ln(j, ljg[...], ljb[...]), wj[...],
                preferred_element_type=jnp.float32) + bj[...]
    o_ref[...] = _ln(j, jtg[...], jtb[...])


def _joint(et, gsum, ljg, ljb, jtg, jtb, wj, bj):
    row = pl.BlockSpec((_RB, _D), lambda i: (i, 0))
    par = pl.BlockSpec((1, _D), lambda i: (0, 0))
    mat = pl.BlockSpec((_D, _D), lambda i: (0, 0))
    return pl.pallas_call(
        _joint_body,
        grid=(_NE // _RB,),
        in_specs=[row, row, par, par, par, par, mat, par],
        out_specs=row,
        out_shape=jax.ShapeDtypeStruct((_NE, _D), jnp.float32),
    )(et, gsum, ljg.reshape(1, _D), ljb.reshape(1, _D), jtg.reshape(1, _D),
      jtb.reshape(1, _D), wj, bj.reshape(1, _D))


def _merge_body(t_ref, p_ref, wa, wb, bm, gg, gb, o_ref):
    n = t_ref.shape[0]
    agg = p_ref[0, 0:n] + p_ref[1, 0:n]
    h = (jnp.dot(t_ref[...], wa[...], preferred_element_type=jnp.float32)
         + jnp.dot(agg, wb[...], preferred_element_type=jnp.float32)
         + bm[...])
    o_ref[...] = t_ref[...] + _ln(jnp.maximum(h, 0.0), gg[...], gb[...])


def _merge(tbl, parts, wa, wb, bm, gg, gb):
    return pl.pallas_call(
        _merge_body,
        out_shape=jax.ShapeDtypeStruct((tbl.shape[0], _D), jnp.float32),
    )(tbl, parts, wa, wb, bm.reshape(1, _D), gg.reshape(1, _D),
      gb.reshape(1, _D))


def _merge2_body(t_ref, p_ref, wa, wb, bm, gg, gb, cg, cb, wr,
                 o_ref, o2_ref):
    n = t_ref.shape[0]
    agg = p_ref[0, 0:n] + p_ref[1, 0:n]
    h = (jnp.dot(t_ref[...], wa[...], preferred_element_type=jnp.float32)
         + jnp.dot(agg, wb[...], preferred_element_type=jnp.float32)
         + bm[...])
    out = t_ref[...] + _ln(jnp.maximum(h, 0.0), gg[...], gb[...])
    o_ref[...] = out
    o2_ref[...] = jnp.dot(_ln(out, cg[...], cb[...]), wr[...],
                          preferred_element_type=jnp.float32)


def _merge2(tbl, parts, wa, wb, bm, gg, gb, cg, cb, wr):
    n = tbl.shape[0]
    return pl.pallas_call(
        _merge2_body,
        out_shape=(jax.ShapeDtypeStruct((n, _D), jnp.float32),
                   jax.ShapeDtypeStruct((n, _D), jnp.float32)),
    )(tbl, parts, wa, wb, bm.reshape(1, _D), gg.reshape(1, _D),
      gb.reshape(1, _D), cg.reshape(1, _D), cb.reshape(1, _D), wr)


@functools.partial(
    pl.kernel,
    out_type=jax.ShapeDtypeStruct((_NE, _D), jnp.float32),
    mesh=_sc_mesh,
    scratch_types=[
        pltpu.VMEM((_CHUNK,), jnp.int32),
        pltpu.VMEM((_CHUNK,), jnp.int32),
        pltpu.VMEM((_CHUNK,), jnp.int32),
        pltpu.VMEM((_CHUNK,), jnp.int32),
        pltpu.VMEM((_CHUNK, _D), jnp.float32),
        pltpu.VMEM((_CHUNK, _D), jnp.float32),
        pltpu.VMEM((_CHUNK, _D), jnp.float32),
        pltpu.VMEM((_CHUNK, _D), jnp.float32),
        pltpu.SemaphoreType.DMA,
        pltpu.SemaphoreType.DMA,
        pltpu.SemaphoreType.DMA,
        pltpu.SemaphoreType.DMA,
        pltpu.SemaphoreType.DMA,
        pltpu.SemaphoreType.DMA,
    ],
)
def _gather_add(vt, ct, eu, ev, out,
                idxu0, idxu1, idxv0, idxv1, a0, a1, b0, b1,
                si0, si1, sg0, sg1, so0, so1):
    sid = lax.axis_index("s")
    wid = sid * _NC + lax.axis_index("c")
    IU, IV, A, B = [idxu0, idxu1], [idxv0, idxv1], [a0, a1], [b0, b1]
    SI, SG, SO = [si0, si1], [sg0, sg1], [so0, so1]

    def base_of(c):
        return pl.multiple_of(wid * _EPW + c * _CHUNK, _CHUNK)

    def fire_idx(c, p):
        b = base_of(c)
        pltpu.async_copy(eu.at[pl.ds(b, _CHUNK)], IU[p], SI[p])
        pltpu.async_copy(ev.at[pl.ds(b, _CHUNK)], IV[p], SI[p])

    def wait_idx(p):
        pltpu.make_async_copy(eu.at[pl.ds(0, _CHUNK)], IU[p], SI[p]).wait()
        pltpu.make_async_copy(ev.at[pl.ds(0, _CHUNK)], IV[p], SI[p]).wait()

    def fire_gather(p):
        pltpu.async_copy(vt.at[IU[p]], A[p], SG[p])
        pltpu.async_copy(ct.at[IV[p]], B[p], SG[p])

    def wait_gather(p):
        pltpu.make_async_copy(vt.at[IU[p]], A[p], SG[p]).wait()
        pltpu.make_async_copy(ct.at[IV[p]], B[p], SG[p]).wait()

    def fire_out(c, p):
        pltpu.async_copy(A[p], out.at[pl.ds(base_of(c), _CHUNK)], SO[p])

    def wait_out(p):
        pltpu.make_async_copy(A[p], out.at[pl.ds(0, _CHUNK)], SO[p]).wait()

    def add(p):
        def addrow(k, carry):
            for u in range(2):
                r = 2 * k + u
                for q in range(_D // 16):
                    s = pl.ds(q * 16, 16)
                    A[p][r, s] = A[p][r, s] + B[p][r, s]
            return carry
        lax.fori_loop(0, _CHUNK // 2, addrow, 0)

    fire_idx(0, 0)
    fire_idx(1, 1)
    wait_idx(0)
    fire_gather(0)
    wait_gather(0); fire_idx(2, 0); wait_idx(1); fire_gather(1)
    add(0); fire_out(0, 0)
    wait_gather(1); fire_idx(3, 1); wait_idx(0); wait_out(0)
    fire_gather(0); add(1); fire_out(1, 1)

    def pair(k, carry):
        c0 = 2 * k
        wait_gather(0); fire_idx(c0 + 2, 0); wait_idx(1); wait_out(1)
        fire_gather(1); add(0); fire_out(c0, 0)
        wait_gather(1); fire_idx(c0 + 3, 1); wait_idx(0); wait_out(0)
        fire_gather(0); add(1); fire_out(c0 + 1, 1)
        return carry

    lax.fori_loop(1, 61, pair, 0)

    wait_gather(0); fire_idx(124, 0); wait_idx(1); wait_out(1)
    fire_gather(1); add(0); fire_out(122, 0)
    wait_gather(1); wait_idx(0); wait_out(0)
    fire_gather(0); add(1); fire_out(123, 1)
    wait_gather(0); add(0); fire_out(124, 0)
    wait_out(1); wait_out(0)


@functools.partial(
    pl.kernel,
    out_type=jax.ShapeDtypeStruct((_NC, _NSEG, _D), jnp.float32),
    mesh=_sc_mesh,
    scratch_types=[
        pltpu.VMEM((_CHUNK,), jnp.int32),
        pltpu.VMEM((_CHUNK,), jnp.int32),
        pltpu.VMEM((_CHUNK, _D), jnp.float32),
        pltpu.VMEM((_CHUNK, _D), jnp.float32),
        pltpu.VMEM((_ZROWS, _D), jnp.float32),
        pltpu.VMEM_SHARED((_NSEG, _D), jnp.float32),
        pltpu.SemaphoreType.DMA,
        pltpu.SemaphoreType.DMA,
        pltpu.SemaphoreType.DMA,
        pltpu.SemaphoreType.DMA,
    ],
)
def _seg_sum(joint, idx, out, i0, i1, r0_, r1_, zbuf, acc, si0, si1, ss0, ss1):
    cid = lax.axis_index("c")
    sid = lax.axis_index("s")
    wid = sid * _NC + cid
    IX, R = [i0, i1], [r0_, r1_]
    SI, SS = [si0, si1], [ss0, ss1]

    def zrow(r, carry):
        for q in range(_D // 16):
            zbuf[r, pl.ds(q * 16, 16)] = jnp.zeros((16,), jnp.float32)
        return carry

    lax.fori_loop(0, _ZROWS, zrow, 0)
    for k in range(_SEG_PT // _ZROWS):
        pltpu.sync_copy(zbuf, acc.at[pl.ds(sid * _SEG_PT + k * _ZROWS, _ZROWS)])
    plsc.subcore_barrier()

    def base_of(c):
        return pl.multiple_of(wid * _EPW + c * _CHUNK, _CHUNK)

    def fire_in(c, p):
        b = base_of(c)
        pltpu.async_copy(idx.at[pl.ds(b, _CHUNK)], IX[p], SI[p])
        pltpu.async_copy(joint.at[pl.ds(b, _CHUNK)], R[p], SI[p])

    def wait_in(p):
        pltpu.make_async_copy(idx.at[pl.ds(0, _CHUNK)], IX[p], SI[p]).wait()
        pltpu.make_async_copy(joint.at[pl.ds(0, _CHUNK)], R[p], SI[p]).wait()

    def fire_scat(p):
        pltpu.async_copy(R[p], acc.at[IX[p]], SS[p], add=True)

    def wait_scat(p):
        pltpu.make_async_copy(R[p], acc.at[IX[p]], SS[p]).wait()

    fire_in(0, 0)
    wait_in(0); fire_scat(0); fire_in(1, 1)

    def pair(k, carry):
        c = 2 * k + 1
        wait_in(1); fire_scat(1); wait_scat(0); fire_in(c + 1, 0)
        wait_in(0); fire_scat(0); wait_scat(1); fire_in(c + 2, 1)
        return carry

    lax.fori_loop(0, 61, pair, 0)

    wait_in(1); fire_scat(1); wait_scat(0); fire_in(124, 0)
    wait_in(0); fire_scat(0); wait_scat(1); wait_scat(0)

    plsc.subcore_barrier()
    for k in range(_SEG_PT // _ZROWS):
        r0 = sid * _SEG_PT + k * _ZROWS
        pltpu.sync_copy(acc.at[pl.ds(r0, _ZROWS)], zbuf)
        pltpu.sync_copy(zbuf, out.at[cid, pl.ds(r0, _ZROWS)])


def kernel(variable_emb, edge_emb, constraint_emb, e_u, e_v, W_left, b_left,
           W_edge, W_right, W_join, b_join, W_merge, b_merge, ln_var_g,
           ln_var_b, ln_edge_g, ln_edge_b, ln_con_g, ln_con_b, ln_joint_g,
           ln_joint_b, ln_join_g, ln_join_b, ln_merge_g, ln_merge_b):
    wm_a = W_merge[:_D]
    wm_b = W_merge[_D:]

    vt, ct1 = _prep(variable_emb, constraint_emb, ln_var_g, ln_var_b,
                    ln_con_g, ln_con_b, W_left, b_left, W_right)
    et = _edge_mm(edge_emb, ln_edge_g, ln_edge_b, W_edge)

    g1 = _gather_add(vt, ct1, e_u, e_v)
    j1 = _joint(et, g1, ln_join_g, ln_join_b, ln_joint_g, ln_joint_b,
                W_join, b_join)
    p1 = _seg_sum(j1, e_v)
    c_new, ct2 = _merge2(ct1, p1, wm_a, wm_b, b_merge, ln_merge_g,
                         ln_merge_b, ln_con_g, ln_con_b, W_right)

    g2 = _gather_add(vt, ct2, e_u, e_v)
    j2 = _joint(et, g2, ln_join_g, ln_join_b, ln_joint_g, ln_joint_b,
                W_join, b_join)
    p2 = _seg_sum(j2, e_u)
    v_new = _merge(vt, p2, wm_a, wm_b, b_merge, ln_merge_g, ln_merge_b)

    return (v_new, c_new)


# speedup vs baseline: 1.2389x; 1.1500x over previous
import functools

import jax
import jax.numpy as jnp
from jax import lax
from jax.experimental import pallas as pl
from jax.experimental.pallas import tpu as pltpu
from jax.experimental.pallas import tpu_sc as plsc

_NU = 10000
_NV = 10000
_NE = 320000
_D = 128

_NC = 2
_NS = 16
_NW = _NC * _NS
_CHUNK = 80
_NSEG = 10240
_SEG_PT = _NSEG // _NS
_ZROWS = 128

_SPLITS = (41, 41, 43)
_E0S = (0, 32 * 41 * _CHUNK, 32 * 82 * _CHUNK)
_ELENS = tuple(32 * n * _CHUNK for n in _SPLITS)

_sc_mesh = plsc.VectorSubcoreMesh(
    core_axis_name="c", subcore_axis_name="s",
    num_cores=_NC, num_subcores=_NS)


def _ln(x, g, b):
    m = jnp.mean(x, axis=-1, keepdims=True)
    d = x - m
    v = jnp.mean(d * d, axis=-1, keepdims=True)
    return d * lax.rsqrt(v + 1e-5) * g + b


def _prep_body(v_ref, c_ref, vg, vb, cg, cb, wl, bl, wr, vt_ref, ct_ref):
    x = _ln(v_ref[...], vg[...], vb[...])
    vt_ref[...] = (
        jnp.dot(x, wl[...], preferred_element_type=jnp.float32) + bl[...])
    y = _ln(c_ref[...], cg[...], cb[...])
    ct_ref[...] = jnp.dot(y, wr[...], preferred_element_type=jnp.float32)


def _prep(v0, c0, vg, vb, cg, cb, wl, bl, wr):
    return pl.pallas_call(
        _prep_body,
        out_shape=(jax.ShapeDtypeStruct((_NU, _D), jnp.float32),
                   jax.ShapeDtypeStruct((_NV, _D), jnp.float32)),
    )(v0, c0, vg.reshape(1, _D), vb.reshape(1, _D), cg.reshape(1, _D),
      cb.reshape(1, _D), wl, bl.reshape(1, _D), wr)


_RB = 8000


def _edge_body(e_ref, leg, leb, we, o_ref):
    o_ref[...] = jnp.dot(_ln(e_ref[...], leg[...], leb[...]), we[...],
                         preferred_element_type=jnp.float32
                         ).astype(jnp.bfloat16)


def _edge_mm(edge_emb, leg, leb, we):
    row = pl.BlockSpec((_RB, _D), lambda i: (i, 0))
    par = pl.BlockSpec((1, _D), lambda i: (0, 0))
    mat = pl.BlockSpec((_D, _D), lambda i: (0, 0))
    return pl.pallas_call(
        _edge_body,
        grid=(_NE // _RB,),
        in_specs=[row, par, par, mat],
        out_specs=row,
        out_shape=jax.ShapeDtypeStruct((_NE, _D), jnp.bfloat16),
    )(edge_emb, leg.reshape(1, _D), leb.reshape(1, _D), we)


def _joint_body(et_ref, g_ref, ljg, ljb, jtg, jtb, wj, bj, o_ref):
    j = jnp.maximum(et_ref[...].astype(jnp.float32) + g_ref[...], 0.0)
    j = jnp.dot(_ln(j, ljg[...], ljb[...]), wj[...],
                preferred_element_type=jnp.float32) + bj[...]
    o_ref[...] = _ln(j, jtg[...], jtb[...])


_JB = 2560


@functools.lru_cache(maxsize=None)
def _joint_call(nrows, off_blocks):
    grid = nrows // _JB
    etrow = pl.BlockSpec((_JB, _D), lambda i: (off_blocks + i, 0))
    row = pl.BlockSpec((_JB, _D), lambda i: (i, 0))
    par = pl.BlockSpec((1, _D), lambda i: (0, 0))
    mat = pl.BlockSpec((_D, _D), lambda i: (0, 0))
    return pl.pallas_call(
        _joint_body,
        grid=(grid,),
        in_specs=[etrow, row, par, par, par, par, mat, par],
        out_specs=row,
        out_shape=jax.ShapeDtypeStruct((nrows, _D), jnp.float32),
    )


def _joint(et, e0, nrows, g_s, ljg, ljb, jtg, jtb, wj, bj):
    return _joint_call(nrows, e0 // _JB)(
        et, g_s, ljg.reshape(1, _D), ljb.reshape(1, _D),
        jtg.reshape(1, _D), jtb.reshape(1, _D), wj, bj.reshape(1, _D))


_MB = 2000


def _merge_body(t_ref, p0, p1, p2, wa, wb, bm, gg, gb, o_ref):
    agg = (p0[0] + p0[1] + p1[0] + p1[1] + p2[0] + p2[1])
    h = (jnp.dot(t_ref[...], wa[...], preferred_element_type=jnp.float32)
         + jnp.dot(agg, wb[...], preferred_element_type=jnp.float32)
         + bm[...])
    o_ref[...] = t_ref[...] + _ln(jnp.maximum(h, 0.0), gg[...], gb[...])


def _merge_specs():
    row = pl.BlockSpec((_MB, _D), lambda i: (i, 0))
    part = pl.BlockSpec((_NC, _MB, _D), lambda i: (0, i, 0))
    par = pl.BlockSpec((1, _D), lambda i: (0, 0))
    mat = pl.BlockSpec((_D, _D), lambda i: (0, 0))
    return row, part, par, mat


def _merge(tbl, parts, wa, wb, bm, gg, gb):
    n = tbl.shape[0]
    row, part, par, mat = _merge_specs()
    return pl.pallas_call(
        _merge_body,
        grid=(n // _MB,),
        in_specs=[row, part, part, part, mat, mat, par, par, par],
        out_specs=row,
        out_shape=jax.ShapeDtypeStruct((n, _D), jnp.float32),
    )(tbl, parts[0], parts[1], parts[2], wa, wb, bm.reshape(1, _D),
      gg.reshape(1, _D), gb.reshape(1, _D))


def _merge2_body(t_ref, p0, p1, p2, wa, wb, bm, gg, gb, cg, cb, wr,
                 o_ref, o2_ref):
    agg = (p0[0] + p0[1] + p1[0] + p1[1] + p2[0] + p2[1])
    h = (jnp.dot(t_ref[...], wa[...], preferred_element_type=jnp.float32)
         + jnp.dot(agg, wb[...], preferred_element_type=jnp.float32)
         + bm[...])
    out = t_ref[...] + _ln(jnp.maximum(h, 0.0), gg[...], gb[...])
    o_ref[...] = out
    o2_ref[...] = jnp.dot(_ln(out, cg[...], cb[...]), wr[...],
                          preferred_element_type=jnp.float32)


def _merge2(tbl, parts, wa, wb, bm, gg, gb, cg, cb, wr):
    n = tbl.shape[0]
    row, part, par, mat = _merge_specs()
    return pl.pallas_call(
        _merge2_body,
        grid=(n // _MB,),
        in_specs=[row, part, part, part, mat, mat, par, par, par, par,
                  par, mat],
        out_specs=(row, row),
        out_shape=(jax.ShapeDtypeStruct((n, _D), jnp.float32),
                   jax.ShapeDtypeStruct((n, _D), jnp.float32)),
    )(tbl, parts[0], parts[1], parts[2], wa, wb, bm.reshape(1, _D),
      gg.reshape(1, _D), gb.reshape(1, _D), cg.reshape(1, _D),
      cb.reshape(1, _D), wr)


@functools.lru_cache(maxsize=None)
def _make_gather(nchunk, e0):
    epw = nchunk * _CHUNK

    @functools.partial(
        pl.kernel,
        out_type=jax.ShapeDtypeStruct((_NW * epw, _D), jnp.float32),
        mesh=_sc_mesh,
        scratch_types=[
            pltpu.VMEM((_CHUNK,), jnp.int32),
            pltpu.VMEM((_CHUNK,), jnp.int32),
            pltpu.VMEM((_CHUNK,), jnp.int32),
            pltpu.VMEM((_CHUNK,), jnp.int32),
            pltpu.VMEM((_CHUNK, _D), jnp.float32),
            pltpu.VMEM((_CHUNK, _D), jnp.float32),
            pltpu.VMEM((_CHUNK, _D), jnp.float32),
            pltpu.VMEM((_CHUNK, _D), jnp.float32),
            pltpu.SemaphoreType.DMA,
            pltpu.SemaphoreType.DMA,
            pltpu.SemaphoreType.DMA,
            pltpu.SemaphoreType.DMA,
            pltpu.SemaphoreType.DMA,
            pltpu.SemaphoreType.DMA,
        ],
    )
    def gather_add(vt, ct, eu, ev, out,
                   idxu0, idxu1, idxv0, idxv1, a0, a1, b0, b1,
                   si0, si1, sg0, sg1, so0, so1):
        wid = lax.axis_index("s") * _NC + lax.axis_index("c")
        IU, IV, A, B = [idxu0, idxu1], [idxv0, idxv1], [a0, a1], [b0, b1]
        SI, SG, SO = [si0, si1], [sg0, sg1], [so0, so1]

        def ibase(c):
            return pl.multiple_of(e0 + wid * epw + c * _CHUNK, _CHUNK)

        def obase(c):
            return pl.multiple_of(wid * epw + c * _CHUNK, _CHUNK)

        def fire_idx(c, p):
            b = ibase(c)
            pltpu.async_copy(eu.at[pl.ds(b, _CHUNK)], IU[p], SI[p])
            pltpu.async_copy(ev.at[pl.ds(b, _CHUNK)], IV[p], SI[p])

        def wait_idx(p):
            pltpu.make_async_copy(eu.at[pl.ds(0, _CHUNK)], IU[p], SI[p]).wait()
            pltpu.make_async_copy(ev.at[pl.ds(0, _CHUNK)], IV[p], SI[p]).wait()

        def fire_gather(p):
            pltpu.async_copy(vt.at[IU[p]], A[p], SG[p])
            pltpu.async_copy(ct.at[IV[p]], B[p], SG[p])

        def wait_gather(p):
            pltpu.make_async_copy(vt.at[IU[p]], A[p], SG[p]).wait()
            pltpu.make_async_copy(ct.at[IV[p]], B[p], SG[p]).wait()

        def fire_out(c, p):
            pltpu.async_copy(A[p], out.at[pl.ds(obase(c), _CHUNK)], SO[p])

        def wait_out(p):
            pltpu.make_async_copy(A[p], out.at[pl.ds(0, _CHUNK)], SO[p]).wait()

        def add(p):
            def addrow(k, carry):
                for u in range(2):
                    r = 2 * k + u
                    for q in range(_D // 16):
                        s = pl.ds(q * 16, 16)
                        A[p][r, s] = A[p][r, s] + B[p][r, s]
                return carry
            lax.fori_loop(0, _CHUNK // 2, addrow, 0)

        fire_idx(0, 0)
        fire_idx(1, 1)
        wait_idx(0)
        fire_gather(0)
        wait_gather(0); fire_idx(2, 0); wait_idx(1); fire_gather(1)
        add(0); fire_out(0, 0)
        wait_gather(1); fire_idx(3, 1); wait_idx(0); wait_out(0)
        fire_gather(0); add(1); fire_out(1, 1)

        def pair(k, carry):
            c0 = 2 * k
            wait_gather(0); fire_idx(c0 + 2, 0); wait_idx(1); wait_out(1)
            fire_gather(1); add(0); fire_out(c0, 0)
            wait_gather(1); fire_idx(c0 + 3, 1); wait_idx(0); wait_out(0)
            fire_gather(0); add(1); fire_out(c0 + 1, 1)
            return carry

        lax.fori_loop(1, (nchunk - 3) // 2, pair, 0)

        wait_gather(0); fire_idx(nchunk - 1, 0); wait_idx(1); wait_out(1)
        fire_gather(1); add(0); fire_out(nchunk - 3, 0)
        wait_gather(1); wait_idx(0); wait_out(0)
        fire_gather(0); add(1); fire_out(nchunk - 2, 1)
        wait_gather(0); add(0); fire_out(nchunk - 1, 0)
        wait_out(1); wait_out(0)

    return gather_add


@functools.lru_cache(maxsize=None)
def _make_seg_sum(nchunk, e0):
    epw = nchunk * _CHUNK

    @functools.partial(
        pl.kernel,
        out_type=jax.ShapeDtypeStruct((_NC, _NSEG, _D), jnp.float32),
        mesh=_sc_mesh,
        scratch_types=[
            pltpu.VMEM((_CHUNK,), jnp.int32),
            pltpu.VMEM((_CHUNK,), jnp.int32),
            pltpu.VMEM((_CHUNK, _D), jnp.float32),
            pltpu.VMEM((_CHUNK, _D), jnp.float32),
            pltpu.VMEM((_ZROWS, _D), jnp.float32),
            pltpu.VMEM_SHARED((_NSEG, _D), jnp.float32),
            pltpu.SemaphoreType.DMA,
            pltpu.SemaphoreType.DMA,
            pltpu.SemaphoreType.DMA,
            pltpu.SemaphoreType.DMA,
        ],
    )
    def seg_sum(joint, idx, out, i0, i1, r0_, r1_, zbuf, acc,
                si0, si1, ss0, ss1):
        cid = lax.axis_index("c")
        sid = lax.axis_index("s")
        wid = sid * _NC + cid
        IX, R = [i0, i1], [r0_, r1_]
        SI, SS = [si0, si1], [ss0, ss1]

        def zrow(r, carry):
            for q in range(_D // 16):
                zbuf[r, pl.ds(q * 16, 16)] = jnp.zeros((16,), jnp.float32)
            return carry

        lax.fori_loop(0, _ZROWS, zrow, 0)
        for k in range(_SEG_PT // _ZROWS):
            pltpu.sync_copy(
                zbuf, acc.at[pl.ds(sid * _SEG_PT + k * _ZROWS, _ZROWS)])
        plsc.subcore_barrier()

        def ibase(c):
            return pl.multiple_of(wid * epw + c * _CHUNK, _CHUNK)

        def gbase(c):
            return pl.multiple_of(e0 + wid * epw + c * _CHUNK, _CHUNK)

        def fire_in(c, p):
            pltpu.async_copy(idx.at[pl.ds(gbase(c), _CHUNK)], IX[p], SI[p])
            pltpu.async_copy(joint.at[pl.ds(ibase(c), _CHUNK)], R[p], SI[p])

        def wait_in(p):
            pltpu.make_async_copy(idx.at[pl.ds(0, _CHUNK)], IX[p], SI[p]).wait()
            pltpu.make_async_copy(
                joint.at[pl.ds(0, _CHUNK)], R[p], SI[p]).wait()

        def fire_scat(p):
            pltpu.async_copy(R[p], acc.at[IX[p]], SS[p], add=True)

        def wait_scat(p):
            pltpu.make_async_copy(R[p], acc.at[IX[p]], SS[p]).wait()

        fire_in(0, 0)
        wait_in(0); fire_scat(0); fire_in(1, 1)

        def pair(k, carry):
            c = 2 * k + 1
            wait_in(1); fire_scat(1); wait_scat(0); fire_in(c + 1, 0)
            wait_in(0); fire_scat(0); wait_scat(1); fire_in(c + 2, 1)
            return carry

        lax.fori_loop(0, (nchunk - 3) // 2, pair, 0)

        wait_in(1); fire_scat(1); wait_scat(0); fire_in(nchunk - 1, 0)
        wait_in(0); fire_scat(0); wait_scat(1); wait_scat(0)

        plsc.subcore_barrier()
        for k in range(_SEG_PT // _ZROWS):
            r0 = sid * _SEG_PT + k * _ZROWS
            pltpu.sync_copy(acc.at[pl.ds(r0, _ZROWS)], zbuf)
            pltpu.sync_copy(zbuf, out.at[cid, pl.ds(r0, _ZROWS)])

    return seg_sum


def _sparse_pass(vt, ct, e_u, e_v, seg_idx, et, ljg, ljb, jtg, jtb, wj, bj):
    parts = []
    for s in range(3):
        n, e0, elen = _SPLITS[s], _E0S[s], _ELENS[s]
        g_s = _make_gather(n, e0)(vt, ct, e_u, e_v)
        j_s = _joint(et, e0, elen, g_s, ljg, ljb, jtg, jtb, wj, bj)
        parts.append(_make_seg_sum(n, e0)(j_s, seg_idx))
    return parts


def kernel(variable_emb, edge_emb, constraint_emb, e_u, e_v, W_left, b_left,
           W_edge, W_right, W_join, b_join, W_merge, b_merge, ln_var_g,
           ln_var_b, ln_edge_g, ln_edge_b, ln_con_g, ln_con_b, ln_joint_g,
           ln_joint_b, ln_join_g, ln_join_b, ln_merge_g, ln_merge_b):
    wm_a = W_merge[:_D]
    wm_b = W_merge[_D:]

    vt, ct1 = _prep(variable_emb, constraint_emb, ln_var_g, ln_var_b,
                    ln_con_g, ln_con_b, W_left, b_left, W_right)
    et = _edge_mm(edge_emb, ln_edge_g, ln_edge_b, W_edge)

    p1 = _sparse_pass(vt, ct1, e_u, e_v, e_v, et, ln_join_g, ln_join_b,
                      ln_joint_g, ln_joint_b, W_join, b_join)
    c_new, ct2 = _merge2(ct1, p1, wm_a, wm_b, b_merge, ln_merge_g,
                         ln_merge_b, ln_con_g, ln_con_b, W_right)

    p2 = _sparse_pass(vt, ct2, e_u, e_v, e_u, et, ln_join_g, ln_join_b,
                      ln_joint_g, ln_joint_b, W_join, b_join)
    v_new = _merge(vt, p2, wm_a, wm_b, b_merge, ln_merge_g, ln_merge_b)

    return (v_new, c_new)


# speedup vs baseline: 1.2485x; 1.0077x over previous
import functools

import jax
import jax.numpy as jnp
from jax import lax
from jax.experimental import pallas as pl
from jax.experimental.pallas import tpu as pltpu
from jax.experimental.pallas import tpu_sc as plsc

_NU = 10000
_NV = 10000
_NE = 320000
_D = 128

_NC = 2
_NS = 16
_NW = _NC * _NS
_CHUNK = 80
_NSEG = 10240
_SEG_PT = _NSEG // _NS
_ZROWS = 128

_SPLITS = (41, 41, 43)
_E0S = (0, 32 * 41 * _CHUNK, 32 * 82 * _CHUNK)
_ELENS = tuple(32 * n * _CHUNK for n in _SPLITS)

_sc_mesh = plsc.VectorSubcoreMesh(
    core_axis_name="c", subcore_axis_name="s",
    num_cores=_NC, num_subcores=_NS)


def _ln(x, g, b):
    m = jnp.mean(x, axis=-1, keepdims=True)
    d = x - m
    v = jnp.mean(d * d, axis=-1, keepdims=True)
    return d * lax.rsqrt(v + 1e-5) * g + b


def _prep_body(v_ref, c_ref, vg, vb, cg, cb, wl, bl, wr, vt_ref, ct_ref):
    x = _ln(v_ref[...], vg[...], vb[...])
    vt_ref[...] = (
        jnp.dot(x, wl[...], preferred_element_type=jnp.float32) + bl[...])
    y = _ln(c_ref[...], cg[...], cb[...])
    ct_ref[...] = jnp.dot(y, wr[...], preferred_element_type=jnp.float32)


def _prep(v0, c0, vg, vb, cg, cb, wl, bl, wr):
    return pl.pallas_call(
        _prep_body,
        out_shape=(jax.ShapeDtypeStruct((_NU, _D), jnp.float32),
                   jax.ShapeDtypeStruct((_NV, _D), jnp.float32)),
    )(v0, c0, vg.reshape(1, _D), vb.reshape(1, _D), cg.reshape(1, _D),
      cb.reshape(1, _D), wl, bl.reshape(1, _D), wr)


_RB = 8000


def _edge_body(e_ref, leg, leb, we, o_ref):
    o_ref[...] = jnp.dot(_ln(e_ref[...], leg[...], leb[...]), we[...],
                         preferred_element_type=jnp.float32
                         ).astype(jnp.bfloat16)


def _edge_mm(edge_emb, leg, leb, we):
    row = pl.BlockSpec((_RB, _D), lambda i: (i, 0))
    par = pl.BlockSpec((1, _D), lambda i: (0, 0))
    mat = pl.BlockSpec((_D, _D), lambda i: (0, 0))
    return pl.pallas_call(
        _edge_body,
        grid=(_NE // _RB,),
        in_specs=[row, par, par, mat],
        out_specs=row,
        out_shape=jax.ShapeDtypeStruct((_NE, _D), jnp.bfloat16),
    )(edge_emb, leg.reshape(1, _D), leb.reshape(1, _D), we)


def _joint_body(et_ref, g_ref, ljg, ljb, jtg, jtb, wj, bj, o_ref):
    j = jnp.maximum(et_ref[...].astype(jnp.float32) + g_ref[...], 0.0)
    j = jnp.dot(_ln(j, ljg[...], ljb[...]), wj[...],
                preferred_element_type=jnp.float32) + bj[...]
    o_ref[...] = _ln(j, jtg[...], jtb[...])


_JB = 2560


@functools.lru_cache(maxsize=None)
def _joint_call(nrows, off_blocks):
    grid = nrows // _JB
    etrow = pl.BlockSpec((_JB, _D), lambda i: (off_blocks + i, 0))
    row = pl.BlockSpec((_JB, _D), lambda i: (i, 0))
    par = pl.BlockSpec((1, _D), lambda i: (0, 0))
    mat = pl.BlockSpec((_D, _D), lambda i: (0, 0))
    return pl.pallas_call(
        _joint_body,
        grid=(grid,),
        in_specs=[etrow, row, par, par, par, par, mat, par],
        out_specs=row,
        out_shape=jax.ShapeDtypeStruct((nrows, _D), jnp.float32),
    )


def _joint(et, e0, nrows, g_s, ljg, ljb, jtg, jtb, wj, bj):
    return _joint_call(nrows, e0 // _JB)(
        et, g_s, ljg.reshape(1, _D), ljb.reshape(1, _D),
        jtg.reshape(1, _D), jtb.reshape(1, _D), wj, bj.reshape(1, _D))


_MB = 2000


def _merge_body(t_ref, p0, p1, p2, wa, wb, bm, gg, gb, o_ref):
    agg = (p0[0] + p0[1] + p1[0] + p1[1] + p2[0] + p2[1])
    h = (jnp.dot(t_ref[...], wa[...], preferred_element_type=jnp.float32)
         + jnp.dot(agg, wb[...], preferred_element_type=jnp.float32)
         + bm[...])
    o_ref[...] = t_ref[...] + _ln(jnp.maximum(h, 0.0), gg[...], gb[...])


def _merge_specs():
    row = pl.BlockSpec((_MB, _D), lambda i: (i, 0))
    part = pl.BlockSpec((_NC, _MB, _D), lambda i: (0, i, 0))
    par = pl.BlockSpec((1, _D), lambda i: (0, 0))
    mat = pl.BlockSpec((_D, _D), lambda i: (0, 0))
    return row, part, par, mat


def _merge(tbl, parts, wa, wb, bm, gg, gb):
    n = tbl.shape[0]
    row, part, par, mat = _merge_specs()
    return pl.pallas_call(
        _merge_body,
        grid=(n // _MB,),
        in_specs=[row, part, part, part, mat, mat, par, par, par],
        out_specs=row,
        out_shape=jax.ShapeDtypeStruct((n, _D), jnp.float32),
    )(tbl, parts[0], parts[1], parts[2], wa, wb, bm.reshape(1, _D),
      gg.reshape(1, _D), gb.reshape(1, _D))


def _merge2_body(t_ref, p0, p1, p2, wa, wb, bm, gg, gb, cg, cb, wr,
                 o_ref, o2_ref):
    agg = (p0[0] + p0[1] + p1[0] + p1[1] + p2[0] + p2[1])
    h = (jnp.dot(t_ref[...], wa[...], preferred_element_type=jnp.float32)
         + jnp.dot(agg, wb[...], preferred_element_type=jnp.float32)
         + bm[...])
    out = t_ref[...] + _ln(jnp.maximum(h, 0.0), gg[...], gb[...])
    o_ref[...] = out
    o2_ref[...] = jnp.dot(_ln(out, cg[...], cb[...]), wr[...],
                          preferred_element_type=jnp.float32)


def _merge2(tbl, parts, wa, wb, bm, gg, gb, cg, cb, wr):
    n = tbl.shape[0]
    row, part, par, mat = _merge_specs()
    return pl.pallas_call(
        _merge2_body,
        grid=(n // _MB,),
        in_specs=[row, part, part, part, mat, mat, par, par, par, par,
                  par, mat],
        out_specs=(row, row),
        out_shape=(jax.ShapeDtypeStruct((n, _D), jnp.float32),
                   jax.ShapeDtypeStruct((n, _D), jnp.float32)),
    )(tbl, parts[0], parts[1], parts[2], wa, wb, bm.reshape(1, _D),
      gg.reshape(1, _D), gb.reshape(1, _D), cg.reshape(1, _D),
      cb.reshape(1, _D), wr)


@functools.lru_cache(maxsize=None)
def _make_gather(nchunk, e0):
    epw = nchunk * _CHUNK

    @functools.partial(
        pl.kernel,
        out_type=jax.ShapeDtypeStruct((_NW * epw, _D), jnp.float32),
        mesh=_sc_mesh,
        scratch_types=[
            pltpu.VMEM((_CHUNK,), jnp.int32),
            pltpu.VMEM((_CHUNK,), jnp.int32),
            pltpu.VMEM((_CHUNK,), jnp.int32),
            pltpu.VMEM((_CHUNK,), jnp.int32),
            pltpu.VMEM((_CHUNK, _D), jnp.float32),
            pltpu.VMEM((_CHUNK, _D), jnp.float32),
            pltpu.VMEM((_CHUNK, _D), jnp.float32),
            pltpu.VMEM((_CHUNK, _D), jnp.float32),
            pltpu.SemaphoreType.DMA,
            pltpu.SemaphoreType.DMA,
            pltpu.SemaphoreType.DMA,
            pltpu.SemaphoreType.DMA,
            pltpu.SemaphoreType.DMA,
            pltpu.SemaphoreType.DMA,
        ],
    )
    def gather_add(vt, ct, eu, ev, out,
                   idxu0, idxu1, idxv0, idxv1, a0, a1, b0, b1,
                   si0, si1, sg0, sg1, so0, so1):
        wid = lax.axis_index("s") * _NC + lax.axis_index("c")
        IU, IV, A, B = [idxu0, idxu1], [idxv0, idxv1], [a0, a1], [b0, b1]
        SI, SG, SO = [si0, si1], [sg0, sg1], [so0, so1]

        def ibase(c):
            return pl.multiple_of(e0 + wid * epw + c * _CHUNK, _CHUNK)

        def obase(c):
            return pl.multiple_of(wid * epw + c * _CHUNK, _CHUNK)

        def fire_idx(c, p):
            b = ibase(c)
            pltpu.async_copy(eu.at[pl.ds(b, _CHUNK)], IU[p], SI[p])
            pltpu.async_copy(ev.at[pl.ds(b, _CHUNK)], IV[p], SI[p])

        def wait_idx(p):
            pltpu.make_async_copy(eu.at[pl.ds(0, _CHUNK)], IU[p], SI[p]).wait()
            pltpu.make_async_copy(ev.at[pl.ds(0, _CHUNK)], IV[p], SI[p]).wait()

        def fire_gather(p):
            pltpu.async_copy(vt.at[IU[p]], A[p], SG[p])
            pltpu.async_copy(ct.at[IV[p]], B[p], SG[p])

        def wait_gather(p):
            pltpu.make_async_copy(vt.at[IU[p]], A[p], SG[p]).wait()
            pltpu.make_async_copy(ct.at[IV[p]], B[p], SG[p]).wait()

        def fire_out(c, p):
            pltpu.async_copy(A[p], out.at[pl.ds(obase(c), _CHUNK)], SO[p])

        def wait_out(p):
            pltpu.make_async_copy(A[p], out.at[pl.ds(0, _CHUNK)], SO[p]).wait()

        def add(p):
            def addrow(k, carry):
                for u in range(2):
                    r = 2 * k + u
                    for q in range(_D // 16):
                        s = pl.ds(q * 16, 16)
                        A[p][r, s] = A[p][r, s] + B[p][r, s]
                return carry
            lax.fori_loop(0, _CHUNK // 2, addrow, 0)

        fire_idx(0, 0)
        fire_idx(1, 1)
        wait_idx(0)
        fire_gather(0)
        wait_gather(0); fire_idx(2, 0); wait_idx(1); fire_gather(1)
        add(0); fire_out(0, 0)
        wait_gather(1); fire_idx(3, 1); wait_idx(0); wait_out(0)
        fire_gather(0); add(1); fire_out(1, 1)

        def pair(k, carry):
            c0 = 2 * k
            wait_gather(0); fire_idx(c0 + 2, 0); wait_idx(1); wait_out(1)
            fire_gather(1); add(0); fire_out(c0, 0)
            wait_gather(1); fire_idx(c0 + 3, 1); wait_idx(0); wait_out(0)
            fire_gather(0); add(1); fire_out(c0 + 1, 1)
            return carry

        lax.fori_loop(1, (nchunk - 3) // 2, pair, 0)

        wait_gather(0); fire_idx(nchunk - 1, 0); wait_idx(1); wait_out(1)
        fire_gather(1); add(0); fire_out(nchunk - 3, 0)
        wait_gather(1); wait_idx(0); wait_out(0)
        fire_gather(0); add(1); fire_out(nchunk - 2, 1)
        wait_gather(0); add(0); fire_out(nchunk - 1, 0)
        wait_out(1); wait_out(0)

    return gather_add


@functools.lru_cache(maxsize=None)
def _make_seg_sum(nchunk, e0):
    epw = nchunk * _CHUNK

    @functools.partial(
        pl.kernel,
        out_type=jax.ShapeDtypeStruct((_NC, _NSEG, _D), jnp.float32),
        mesh=_sc_mesh,
        scratch_types=[
            pltpu.VMEM((_CHUNK,), jnp.int32),
            pltpu.VMEM((_CHUNK,), jnp.int32),
            pltpu.VMEM((_CHUNK, _D), jnp.float32),
            pltpu.VMEM((_CHUNK, _D), jnp.float32),
            pltpu.VMEM((_ZROWS, _D), jnp.float32),
            pltpu.VMEM_SHARED((_NSEG, _D), jnp.float32),
            pltpu.SemaphoreType.DMA,
            pltpu.SemaphoreType.DMA,
            pltpu.SemaphoreType.DMA,
            pltpu.SemaphoreType.DMA,
        ],
    )
    def seg_sum(joint, idx, out, i0, i1, r0_, r1_, zbuf, acc,
                si0, si1, ss0, ss1):
        cid = lax.axis_index("c")
        sid = lax.axis_index("s")
        wid = sid * _NC + cid
        IX, R = [i0, i1], [r0_, r1_]
        SI, SS = [si0, si1], [ss0, ss1]

        def ibase(c):
            return pl.multiple_of(wid * epw + c * _CHUNK, _CHUNK)

        def gbase(c):
            return pl.multiple_of(e0 + wid * epw + c * _CHUNK, _CHUNK)

        def fire_in(c, p):
            pltpu.async_copy(idx.at[pl.ds(gbase(c), _CHUNK)], IX[p], SI[p])
            pltpu.async_copy(joint.at[pl.ds(ibase(c), _CHUNK)], R[p], SI[p])

        fire_in(0, 0)
        fire_in(1, 1)

        def zrow(r, carry):
            for q in range(_D // 16):
                zbuf[r, pl.ds(q * 16, 16)] = jnp.zeros((16,), jnp.float32)
            return carry

        lax.fori_loop(0, _ZROWS, zrow, 0)
        for k in range(_SEG_PT // _ZROWS):
            pltpu.sync_copy(
                zbuf, acc.at[pl.ds(sid * _SEG_PT + k * _ZROWS, _ZROWS)])
        plsc.subcore_barrier()

        def wait_in(p):
            pltpu.make_async_copy(idx.at[pl.ds(0, _CHUNK)], IX[p], SI[p]).wait()
            pltpu.make_async_copy(
                joint.at[pl.ds(0, _CHUNK)], R[p], SI[p]).wait()

        def fire_scat(p):
            pltpu.async_copy(R[p], acc.at[IX[p]], SS[p], add=True)

        def wait_scat(p):
            pltpu.make_async_copy(R[p], acc.at[IX[p]], SS[p]).wait()

        wait_in(0); fire_scat(0)

        def pair(k, carry):
            c = 2 * k + 1
            wait_in(1); fire_scat(1); wait_scat(0); fire_in(c + 1, 0)
            wait_in(0); fire_scat(0); wait_scat(1); fire_in(c + 2, 1)
            return carry

        lax.fori_loop(0, (nchunk - 3) // 2, pair, 0)

        wait_in(1); fire_scat(1); wait_scat(0); fire_in(nchunk - 1, 0)
        wait_in(0); fire_scat(0); wait_scat(1); wait_scat(0)

        plsc.subcore_barrier()
        for k in range(_SEG_PT // _ZROWS):
            r0 = sid * _SEG_PT + k * _ZROWS
            pltpu.sync_copy(acc.at[pl.ds(r0, _ZROWS)], zbuf)
            pltpu.sync_copy(zbuf, out.at[cid, pl.ds(r0, _ZROWS)])

    return seg_sum


def _sparse_pass(vt, ct, e_u, e_v, seg_idx, et, ljg, ljb, jtg, jtb, wj, bj):
    parts = []
    for s in range(3):
        n, e0, elen = _SPLITS[s], _E0S[s], _ELENS[s]
        g_s = _make_gather(n, e0)(vt, ct, e_u, e_v)
        j_s = _joint(et, e0, elen, g_s, ljg, ljb, jtg, jtb, wj, bj)
        parts.append(_make_seg_sum(n, e0)(j_s, seg_idx))
    return parts


def kernel(variable_emb, edge_emb, constraint_emb, e_u, e_v, W_left, b_left,
           W_edge, W_right, W_join, b_join, W_merge, b_merge, ln_var_g,
           ln_var_b, ln_edge_g, ln_edge_b, ln_con_g, ln_con_b, ln_joint_g,
           ln_joint_b, ln_join_g, ln_join_b, ln_merge_g, ln_merge_b):
    wm_a = W_merge[:_D]
    wm_b = W_merge[_D:]

    vt, ct1 = _prep(variable_emb, constraint_emb, ln_var_g, ln_var_b,
                    ln_con_g, ln_con_b, W_left, b_left, W_right)
    et = _edge_mm(edge_emb, ln_edge_g, ln_edge_b, W_edge)

    p1 = _sparse_pass(vt, ct1, e_u, e_v, e_v, et, ln_join_g, ln_join_b,
                      ln_joint_g, ln_joint_b, W_join, b_join)
    c_new, ct2 = _merge2(ct1, p1, wm_a, wm_b, b_merge, ln_merge_g,
                         ln_merge_b, ln_con_g, ln_con_b, W_right)

    p2 = _sparse_pass(vt, ct2, e_u, e_v, e_u, et, ln_join_g, ln_join_b,
                      ln_joint_g, ln_joint_b, W_join, b_join)
    v_new = _merge(vt, p2, wm_a, wm_b, b_merge, ln_merge_g, ln_merge_b)

    return (v_new, c_new)
